# baseline (speedup 1.0000x reference)
import functools

import jax
import jax.numpy as jnp
from jax import lax
from jax.experimental import pallas as pl
from jax.experimental.pallas import tpu as pltpu
from jax.experimental.pallas import tpu_sc as plsc

BATCH = 16384
FIELDS = 26
FPAD = 32
EMBED_DIM = 32
NC, NS = 2, 16
NW = NC * NS
BPW = BATCH // NW
NB = 32
STEPS = BPW // NB
IDX_COLS = 128
B_PER_ROW = IDX_COLS // FPAD
IDX_ROWS = BATCH // B_PER_ROW
IRPW = BPW // B_PER_ROW


def _sc_gather(idx, table):
    mesh = plsc.VectorSubcoreMesh(
        core_axis_name="c", subcore_axis_name="s",
        num_cores=NC, num_subcores=NS)

    @functools.partial(
        pl.kernel,
        out_type=jax.ShapeDtypeStruct((BATCH, FIELDS, EMBED_DIM), jnp.float32),
        mesh=mesh,
        scratch_types=[
            pltpu.VMEM((IRPW, IDX_COLS), jnp.int32),
            pltpu.VMEM((NB, FPAD, EMBED_DIM), jnp.float32),
            pltpu.VMEM((NB, FPAD, EMBED_DIM), jnp.float32),
            pltpu.SemaphoreType.DMA,
            pltpu.SemaphoreType.DMA,
            pltpu.SemaphoreType.DMA,
            pltpu.SemaphoreType.DMA,
        ],
        compiler_params=pltpu.CompilerParams(use_tc_tiling_on_sc=False),
    )
    def k(idx_hbm, table_hbm, out_hbm, idx_v, rows0, rows1, sg0, sg1, sw0, sw1):
        wid = lax.axis_index("s") * NC + lax.axis_index("c")
        b0 = wid * BPW
        pltpu.sync_copy(idx_hbm.at[pl.ds(wid * IRPW, IRPW)], idx_v)
        rows = (rows0, rows1)
        sg = (sg0, sg1)
        sw = (sw0, sw1)

        def fire_g(b, t):
            for s in range(NB):
                row = t * (NB // B_PER_ROW) + s // B_PER_ROW
                col = (s % B_PER_ROW) * FPAD
                pltpu.async_copy(
                    table_hbm.at[idx_v.at[row, pl.ds(col, FPAD)]],
                    rows[b].at[s], sg[b])

        def drain_g(b):
            for s in range(NB):
                pltpu.make_async_copy(
                    table_hbm.at[idx_v.at[0, pl.ds(0, FPAD)]],
                    rows[b].at[s], sg[b]).wait()

        def fire_w(b, t):
            pltpu.async_copy(
                rows[b].at[pl.ds(0, NB), pl.ds(0, FIELDS)],
                out_hbm.at[pl.ds(b0 + t * NB, NB)], sw[b])

        def wait_w(b):
            pltpu.make_async_copy(
                rows[b].at[pl.ds(0, NB), pl.ds(0, FIELDS)],
                out_hbm.at[pl.ds(b0, NB)], sw[b]).wait()

        fire_g(0, 0)
        fire_g(1, 1)

        @pl.loop(0, STEPS // 2 - 1)
        def body(i):
            t = i * 2
            drain_g(0)
            fire_w(0, t)
            drain_g(1)
            fire_w(1, t + 1)
            wait_w(0)
            fire_g(0, t + 2)
            wait_w(1)
            fire_g(1, t + 3)

        drain_g(0)
        fire_w(0, STEPS - 2)
        drain_g(1)
        fire_w(1, STEPS - 1)
        wait_w(0)
        wait_w(1)

    return k(idx, table)


def kernel(inputs, table):
    idx = jnp.pad(inputs.astype(jnp.int32), ((0, 0), (0, FPAD - FIELDS)))
    idx = idx.reshape(IDX_ROWS, IDX_COLS)
    return _sc_gather(idx, table)


# speedup vs baseline: 2.2802x; 2.2802x over previous
import functools

import jax
import jax.numpy as jnp
from jax import lax
from jax.experimental import pallas as pl
from jax.experimental.pallas import tpu as pltpu
from jax.experimental.pallas import tpu_sc as plsc

VOCAB = 1000000
BATCH = 16384
FIELDS = 26
EMBED_DIM = 32
NC, NS = 2, 16
NW = NC * NS
BPW = BATCH // NW
NB = 32
STEPS = BPW // NB


def _sc_gather(idx, table):
    mesh = plsc.VectorSubcoreMesh(
        core_axis_name="c", subcore_axis_name="s",
        num_cores=NC, num_subcores=NS)

    @functools.partial(
        pl.kernel,
        out_type=jax.ShapeDtypeStruct((BATCH, FIELDS, EMBED_DIM), jnp.float32),
        mesh=mesh,
        scratch_types=[
            pltpu.VMEM((BPW, FIELDS), jnp.int32),
            pltpu.VMEM((NB, FIELDS, EMBED_DIM), jnp.float32),
            pltpu.VMEM((NB, FIELDS, EMBED_DIM), jnp.float32),
            pltpu.SemaphoreType.DMA,
            pltpu.SemaphoreType.DMA,
            pltpu.SemaphoreType.DMA,
            pltpu.SemaphoreType.DMA,
        ],
        compiler_params=pltpu.CompilerParams(use_tc_tiling_on_sc=False),
    )
    def k(idx_hbm, table_hbm, out_hbm, idx_v, rows0, rows1, sg0, sg1, sw0, sw1):
        wid = lax.axis_index("s") * NC + lax.axis_index("c")
        b0 = wid * BPW
        pltpu.sync_copy(idx_hbm.at[pl.ds(b0, BPW)], idx_v)
        rows = (rows0, rows1)
        sg = (sg0, sg1)
        sw = (sw0, sw1)

        def fire_g(b, t):
            for s in range(NB):
                pltpu.async_copy(
                    table_hbm.at[idx_v.at[t * NB + s]], rows[b].at[s], sg[b])

        def drain_g(b):
            for s in range(NB):
                pltpu.make_async_copy(
                    table_hbm.at[idx_v.at[s]], rows[b].at[s], sg[b]).wait()

        def fire_w(b, t):
            pltpu.async_copy(
                rows[b], out_hbm.at[pl.ds(b0 + t * NB, NB)], sw[b])

        def wait_w(b):
            pltpu.make_async_copy(
                rows[b], out_hbm.at[pl.ds(b0, NB)], sw[b]).wait()

        fire_g(0, 0)
        fire_g(1, 1)

        @pl.loop(0, STEPS // 2 - 1)
        def body(i):
            t = i * 2
            drain_g(0)
            fire_w(0, t)
            drain_g(1)
            fire_w(1, t + 1)
            wait_w(0)
            fire_g(0, t + 2)
            wait_w(1)
            fire_g(1, t + 3)

        drain_g(0)
        fire_w(0, STEPS - 2)
        drain_g(1)
        fire_w(1, STEPS - 1)
        wait_w(0)
        wait_w(1)

    return k(idx, table)


def kernel(inputs, table):
    table_flat = jax.lax.optimization_barrier(table.reshape(VOCAB * EMBED_DIM))
    table2 = table_flat.reshape(VOCAB, EMBED_DIM)
    return _sc_gather(inputs.astype(jnp.int32), table2)
